# baseline (speedup 1.0000x reference)
import functools

import jax
import jax.numpy as jnp
from jax import lax
from jax.experimental import pallas as pl
from jax.experimental.pallas import tpu as pltpu
from jax.experimental.pallas import tpu_sc as plsc

D = 128
NC = 2
NS = 16
NW = NC * NS
CH = 128
RING = 2
ROWB = 1024


def _pads(n, e):
    rpt = (-(-n // NS) + 7) // 8 * 8
    npad = rpt * NS
    nch = -(-e // (NW * CH))
    nch = -(-nch // RING) * RING
    return rpt, npad, nch


def _make_seg_sum(n, e, with_cnt):
    rpt, npad, nch = _pads(n, e)
    nz = rpt // 16

    mesh = plsc.VectorSubcoreMesh(
        core_axis_name="c", subcore_axis_name="s",
        num_cores=NC, num_subcores=NS)

    out_type = [jax.ShapeDtypeStruct((NC, npad, D), jnp.float32)]
    scratch = (
        [pltpu.VMEM((CH,), jnp.int32)] * (2 * RING)
        + [pltpu.VMEM((CH, D), jnp.float32)] * RING
        + [
            pltpu.VMEM((16, D), jnp.float32),
            pltpu.VMEM_SHARED((npad, D), jnp.float32),
        ]
        + [pltpu.SemaphoreType.DMA] * RING
        + [pltpu.SemaphoreType.DMA]
    )
    if with_cnt:
        out_type.append(jax.ShapeDtypeStruct((NW, npad), jnp.float32))
        scratch.append(pltpu.VMEM((npad,), jnp.float32))

    @functools.partial(
        pl.kernel, out_type=tuple(out_type), mesh=mesh,
        scratch_types=tuple(scratch),
        compiler_params=pltpu.CompilerParams(needs_layout_passes=False))
    def seg_sum(t_hbm, src_hbm, dst_hbm, out_hbm, *rest):
        if with_cnt:
            cnt_hbm, rest = rest[0], rest[1:]
            cnt_v = rest[-1]
            rest = rest[:-1]
        sbufs = rest[0:2 * RING:2]
        dbufs = rest[1:2 * RING:2]
        rbufs = rest[2 * RING:3 * RING]
        zbuf, acc = rest[3 * RING], rest[3 * RING + 1]
        sems = rest[3 * RING + 2:4 * RING + 2]
        semz = rest[4 * RING + 2]
        cid = lax.axis_index("c")
        sid = lax.axis_index("s")
        wid = cid * NS + sid

        zero16 = jnp.zeros((16,), jnp.float32)
        for r in range(16):
            for cc in range(D // 16):
                zbuf[r, pl.ds(cc * 16, 16)] = zero16

        ebase = wid * (nch * CH)

        zbase = sid * rpt
        zd = [pltpu.async_copy(zbuf, acc.at[pl.ds(zbase + k * 16, 16)], semz)
              for k in range(nz)]
        zrem = rpt - nz * 16
        if zrem:
            zd.append(pltpu.async_copy(
                zbuf.at[pl.ds(0, zrem)],
                acc.at[pl.ds(zbase + nz * 16, zrem)], semz))

        def load(j, b):
            off = ebase + j * CH
            pltpu.sync_copy(src_hbm.at[pl.ds(off, CH)], sbufs[b])
            pltpu.sync_copy(dst_hbm.at[pl.ds(off, CH)], dbufs[b])
            pltpu.async_copy(t_hbm.at[sbufs[b]], rbufs[b], sems[b])

        def drain(b):
            pltpu.make_async_copy(
                t_hbm.at[pl.ds(0, CH)], rbufs[b], sems[b]).wait()

        for b in range(RING):
            load(b, b)

        if with_cnt:
            def czloop(j, carry):
                cnt_v[pl.ds(j * 16, 16)] = zero16
                return carry
            lax.fori_loop(0, npad // 16, czloop, 0)
            ones16 = jnp.ones((16,), jnp.float32)

        def consume(b):
            pltpu.sync_copy(rbufs[b], acc.at[dbufs[b]], add=True)
            if with_cnt:
                for k in range(CH // 16):
                    idx16 = dbufs[b][pl.ds(k * 16, 16)]
                    plsc.addupdate_scatter(cnt_v, [idx16], ones16)

        for d in zd:
            d.wait()
        plsc.subcore_barrier()

        def body(i, carry):
            for b in range(RING):
                drain(b)
                consume(b)
                load(RING * i + b + RING, b)
            return carry

        lax.fori_loop(0, nch // RING - 1, body, 0)
        for b in range(RING):
            drain(b)
            consume(b)

        plsc.subcore_barrier()
        pltpu.sync_copy(acc.at[pl.ds(zbase, rpt)],
                        out_hbm.at[cid, pl.ds(zbase, rpt)])
        if with_cnt:
            pltpu.sync_copy(cnt_v, cnt_hbm.at[wid])

    return seg_sum, npad


def _row_spec(w):
    return pl.BlockSpec((ROWB, w), lambda i: (i, 0))


def _full_spec(shape):
    return pl.BlockSpec(shape, lambda i: tuple(0 for _ in shape))


def _lin0_body(x_ref, w_ref, o_ref):
    o_ref[...] = jnp.dot(x_ref[...], w_ref[...],
                         preferred_element_type=jnp.float32)


def _sage_body(a0_ref, a1_ref, cnt_ref, t_ref, wl_ref, wr_ref, b_ref, o_ref):
    s = a0_ref[...] + a1_ref[...]
    cnt = jnp.maximum(jnp.sum(cnt_ref[...], axis=0), 1.0)[:, None]
    mean = s / cnt
    o_ref[...] = (jnp.dot(mean, wl_ref[...], preferred_element_type=jnp.float32)
                  + jnp.dot(t_ref[...], wr_ref[...],
                            preferred_element_type=jnp.float32)
                  + b_ref[...])


def _final_body(a0_ref, a1_ref, cnt_ref, t_ref, c_ref, i_ref, f_ref,
                wl_ref, wr_ref, b_ref, o_ref):
    s = a0_ref[...] + a1_ref[...]
    cnt = jnp.maximum(jnp.sum(cnt_ref[...], axis=0), 1.0)[:, None]
    mean = s / cnt
    t2 = (jnp.dot(mean, wl_ref[...], preferred_element_type=jnp.float32)
          + jnp.dot(t_ref[...], wr_ref[...],
                    preferred_element_type=jnp.float32)
          + b_ref[...])
    o_ref[...] = f_ref[...] * c_ref[...] + i_ref[...] * jnp.tanh(t2)


def kernel(x, edge_index, h, c, i, f, W_lin, b_node, Wl1, Wr1, b1, Wl2, Wr2, b2):
    n = x.shape[0]
    e = edge_index.shape[1]
    grid = (-(-n // ROWB),)

    rpt, npad, nch = _pads(n, e)
    e_pad = NW * nch * CH
    ar = jnp.arange(e_pad - e, dtype=jnp.int32)
    src3 = jnp.concatenate([edge_index[0], ar % jnp.int32(n)])
    dst3 = jnp.concatenate(
        [edge_index[1], jnp.int32(n) + ar % jnp.int32(npad - n)])

    seg_sum1, _ = _make_seg_sum(n, e, with_cnt=True)
    seg_sum2, _ = _make_seg_sum(n, e, with_cnt=False)

    t0 = pl.pallas_call(
        _lin0_body,
        grid=grid,
        in_specs=[_row_spec(D), _full_spec((D, D))],
        out_specs=_row_spec(D),
        out_shape=jax.ShapeDtypeStruct((n, D), jnp.float32),
    )(x, W_lin.T)

    agg1, cnt = seg_sum1(t0, src3, dst3)

    cnt_spec = pl.BlockSpec((NW, ROWB), lambda i: (0, i))

    t1 = pl.pallas_call(
        _sage_body,
        grid=grid,
        in_specs=[_row_spec(D), _row_spec(D), cnt_spec, _row_spec(D),
                  _full_spec((D, D)), _full_spec((D, D)), _full_spec((1, D))],
        out_specs=_row_spec(D),
        out_shape=jax.ShapeDtypeStruct((n, D), jnp.float32),
    )(agg1[0], agg1[1], cnt, t0, Wl1.T, Wr1.T, b1.reshape(1, D))

    (agg2,) = seg_sum2(t1, src3, dst3)

    out = pl.pallas_call(
        _final_body,
        grid=grid,
        in_specs=[_row_spec(D), _row_spec(D), cnt_spec, _row_spec(D),
                  _row_spec(D), _row_spec(D), _row_spec(D),
                  _full_spec((D, D)), _full_spec((D, D)), _full_spec((1, D))],
        out_specs=_row_spec(D),
        out_shape=jax.ShapeDtypeStruct((n, D), jnp.float32),
    )(agg2[0], agg2[1], cnt, t1, c, i, f,
      Wl2.T, Wr2.T, (b2 + b_node).reshape(1, D))

    return out


# speedup vs baseline: 1.3082x; 1.3082x over previous
import functools

import jax
import jax.numpy as jnp
from jax import lax
from jax.experimental import pallas as pl
from jax.experimental.pallas import tpu as pltpu
from jax.experimental.pallas import tpu_sc as plsc

D = 128
NC = 2
NS = 16
NW = NC * NS
CH = 128
RING = 2
ROWB = 1024


def _pads(n, e):
    rpt = (-(-n // NS) + 7) // 8 * 8
    npad = rpt * NS
    nch = -(-e // (NW * CH))
    nch = -(-nch // (2 * RING)) * (2 * RING)
    return rpt, npad, nch


def _make_seg_sum(n, e, with_cnt):
    rpt, npad, nch = _pads(n, e)
    nz = rpt // 16

    mesh = plsc.VectorSubcoreMesh(
        core_axis_name="c", subcore_axis_name="s",
        num_cores=NC, num_subcores=NS)

    IR = 2 * RING
    out_type = [jax.ShapeDtypeStruct((NC, npad, D), jnp.float32)]
    scratch = (
        [pltpu.VMEM((CH,), jnp.int32)] * (2 * IR)
        + [pltpu.VMEM((CH, D), jnp.float32)] * RING
        + [
            pltpu.VMEM((16, D), jnp.float32),
            pltpu.VMEM_SHARED((npad, D), jnp.float32),
        ]
        + [pltpu.SemaphoreType.DMA] * RING
        + [pltpu.SemaphoreType.DMA] * IR
        + [pltpu.SemaphoreType.DMA]
    )
    if with_cnt:
        out_type.append(jax.ShapeDtypeStruct((NW, npad), jnp.float32))
        scratch.append(pltpu.VMEM((npad,), jnp.float32))

    @functools.partial(
        pl.kernel, out_type=tuple(out_type), mesh=mesh,
        scratch_types=tuple(scratch),
        compiler_params=pltpu.CompilerParams(needs_layout_passes=False))
    def seg_sum(t_hbm, src_hbm, dst_hbm, out_hbm, *rest):
        if with_cnt:
            cnt_hbm, rest = rest[0], rest[1:]
            cnt_v = rest[-1]
            rest = rest[:-1]
        sbufs = rest[0:2 * IR:2]
        dbufs = rest[1:2 * IR:2]
        rbufs = rest[2 * IR:2 * IR + RING]
        zbuf, acc = rest[2 * IR + RING], rest[2 * IR + RING + 1]
        sems = rest[2 * IR + RING + 2:2 * IR + 2 * RING + 2]
        semi = rest[2 * IR + 2 * RING + 2:2 * IR + 2 * RING + 2 + IR]
        semz = rest[2 * IR + 2 * RING + 2 + IR]
        cid = lax.axis_index("c")
        sid = lax.axis_index("s")
        wid = cid * NS + sid

        zero16 = jnp.zeros((16,), jnp.float32)
        for r in range(16):
            for cc in range(D // 16):
                zbuf[r, pl.ds(cc * 16, 16)] = zero16

        ebase = wid * (nch * CH)

        zbase = sid * rpt
        zd = [pltpu.async_copy(zbuf, acc.at[pl.ds(zbase + k * 16, 16)], semz)
              for k in range(nz)]
        zrem = rpt - nz * 16
        if zrem:
            zd.append(pltpu.async_copy(
                zbuf.at[pl.ds(0, zrem)],
                acc.at[pl.ds(zbase + nz * 16, zrem)], semz))

        def idx_load(j, q):
            off = ebase + j * CH
            pltpu.async_copy(src_hbm.at[pl.ds(off, CH)], sbufs[q], semi[q])
            pltpu.async_copy(dst_hbm.at[pl.ds(off, CH)], dbufs[q], semi[q])

        def idx_wait(q):
            pltpu.make_async_copy(
                src_hbm.at[pl.ds(0, CH)], sbufs[q], semi[q]).wait()
            pltpu.make_async_copy(
                dst_hbm.at[pl.ds(0, CH)], dbufs[q], semi[q]).wait()

        def gather(q, b):
            idx_wait(q)
            pltpu.async_copy(t_hbm.at[sbufs[q]], rbufs[b], sems[b])

        def drain(b):
            pltpu.make_async_copy(
                t_hbm.at[pl.ds(0, CH)], rbufs[b], sems[b]).wait()

        for q in range(IR):
            idx_load(q, q)
        for b in range(RING):
            gather(b, b)

        if with_cnt:
            def czloop(j, carry):
                cnt_v[pl.ds(j * 16, 16)] = zero16
                return carry
            lax.fori_loop(0, npad // 16, czloop, 0)
            ones16 = jnp.ones((16,), jnp.float32)

        def consume(b, q):
            pltpu.sync_copy(rbufs[b], acc.at[dbufs[q]], add=True)
            if with_cnt:
                for k in range(CH // 16):
                    idx16 = dbufs[q][pl.ds(k * 16, 16)]
                    plsc.addupdate_scatter(cnt_v, [idx16], ones16)

        for d in zd:
            d.wait()
        plsc.subcore_barrier()

        def body(i, carry):
            for s in range(IR):
                b = s % RING
                drain(b)
                consume(b, s)
                idx_load(IR * i + s + IR, s)
                gather((s + RING) % IR, b)
            return carry

        lax.fori_loop(0, nch // IR - 1, body, 0)
        for s in range(IR):
            b = s % RING
            drain(b)
            consume(b, s)
            if s < IR - RING:
                gather((s + RING) % IR, b)

        plsc.subcore_barrier()
        pltpu.sync_copy(acc.at[pl.ds(zbase, rpt)],
                        out_hbm.at[cid, pl.ds(zbase, rpt)])
        if with_cnt:
            pltpu.sync_copy(cnt_v, cnt_hbm.at[wid])

    return seg_sum, npad


def _row_spec(w):
    return pl.BlockSpec((ROWB, w), lambda i: (i, 0))


def _full_spec(shape):
    return pl.BlockSpec(shape, lambda i: tuple(0 for _ in shape))


def _lin0_body(x_ref, w_ref, o_ref):
    o_ref[...] = jnp.dot(x_ref[...], w_ref[...],
                         preferred_element_type=jnp.float32)


def _sage_body(a0_ref, a1_ref, cnt_ref, t_ref, wl_ref, wr_ref, b_ref, o_ref):
    s = a0_ref[...] + a1_ref[...]
    cnt = jnp.maximum(jnp.sum(cnt_ref[...], axis=0), 1.0)[:, None]
    mean = s / cnt
    o_ref[...] = (jnp.dot(mean, wl_ref[...], preferred_element_type=jnp.float32)
                  + jnp.dot(t_ref[...], wr_ref[...],
                            preferred_element_type=jnp.float32)
                  + b_ref[...])


def _final_body(a0_ref, a1_ref, cnt_ref, t_ref, c_ref, i_ref, f_ref,
                wl_ref, wr_ref, b_ref, o_ref):
    s = a0_ref[...] + a1_ref[...]
    cnt = jnp.maximum(jnp.sum(cnt_ref[...], axis=0), 1.0)[:, None]
    mean = s / cnt
    t2 = (jnp.dot(mean, wl_ref[...], preferred_element_type=jnp.float32)
          + jnp.dot(t_ref[...], wr_ref[...],
                    preferred_element_type=jnp.float32)
          + b_ref[...])
    o_ref[...] = f_ref[...] * c_ref[...] + i_ref[...] * jnp.tanh(t2)


def kernel(x, edge_index, h, c, i, f, W_lin, b_node, Wl1, Wr1, b1, Wl2, Wr2, b2):
    n = x.shape[0]
    e = edge_index.shape[1]
    grid = (-(-n // ROWB),)

    rpt, npad, nch = _pads(n, e)
    e_pad = NW * nch * CH
    ar = jnp.arange(e_pad - e, dtype=jnp.int32)
    src3 = jnp.concatenate([edge_index[0], ar % jnp.int32(n)])
    dst3 = jnp.concatenate(
        [edge_index[1], jnp.int32(n) + ar % jnp.int32(npad - n)])

    seg_sum1, _ = _make_seg_sum(n, e, with_cnt=True)
    seg_sum2, _ = _make_seg_sum(n, e, with_cnt=False)

    t0 = pl.pallas_call(
        _lin0_body,
        grid=grid,
        in_specs=[_row_spec(D), _full_spec((D, D))],
        out_specs=_row_spec(D),
        out_shape=jax.ShapeDtypeStruct((n, D), jnp.float32),
    )(x, W_lin.T)

    agg1, cnt = seg_sum1(t0, src3, dst3)

    cnt_spec = pl.BlockSpec((NW, ROWB), lambda i: (0, i))

    t1 = pl.pallas_call(
        _sage_body,
        grid=grid,
        in_specs=[_row_spec(D), _row_spec(D), cnt_spec, _row_spec(D),
                  _full_spec((D, D)), _full_spec((D, D)), _full_spec((1, D))],
        out_specs=_row_spec(D),
        out_shape=jax.ShapeDtypeStruct((n, D), jnp.float32),
    )(agg1[0], agg1[1], cnt, t0, Wl1.T, Wr1.T, b1.reshape(1, D))

    (agg2,) = seg_sum2(t1, src3, dst3)

    out = pl.pallas_call(
        _final_body,
        grid=grid,
        in_specs=[_row_spec(D), _row_spec(D), cnt_spec, _row_spec(D),
                  _row_spec(D), _row_spec(D), _row_spec(D),
                  _full_spec((D, D)), _full_spec((D, D)), _full_spec((1, D))],
        out_specs=_row_spec(D),
        out_shape=jax.ShapeDtypeStruct((n, D), jnp.float32),
    )(agg2[0], agg2[1], cnt, t1, c, i, f,
      Wl2.T, Wr2.T, (b2 + b_node).reshape(1, D))

    return out
